# Initial kernel scaffold; baseline (speedup 1.0000x reference)
import jax
import jax.numpy as jnp
from jax.experimental import pallas as pl


def kernel(h, edge_index):
    raise NotImplementedError("write your pallas kernel here")


# speedup vs baseline: 2.9144x; 2.9144x over previous
import functools

import jax
import jax.numpy as jnp
from jax import lax
from jax.experimental import pallas as pl
from jax.experimental.pallas import tpu as pltpu
from jax.experimental.pallas import tpu_sc as plsc

ALPHA = 0.5
N_NODES = 10000
N_EDGES = 320000
D = 128

NC = 2
NS = 16
NW = NC * NS
CHUNK = 128
E_PAD = 327680
TOTAL_CHUNKS = E_PAD // CHUNK
CHUNKS_PER_W = TOTAL_CHUNKS // NW
ACC_ROWS = 10240
ZROWS = ACC_ROWS // NS


def _sc_scatter_body(h_hbm, src_hbm, dst_hbm, zeros_hbm, out_hbm,
                     src_v, dst_v, rows_v, acc, sem):
    c = lax.axis_index("c")
    s = lax.axis_index("s")
    wid = s * NC + c

    pltpu.sync_copy(zeros_hbm, acc.at[pl.ds(s * ZROWS, ZROWS)])
    plsc.subcore_barrier()

    base = wid * CHUNKS_PER_W

    def chunk_body(j, carry):
        cidx = base + j
        pltpu.sync_copy(src_hbm.at[cidx], src_v)
        pltpu.sync_copy(dst_hbm.at[cidx], dst_v)
        pltpu.async_copy(h_hbm.at[src_v], rows_v, sem).wait()
        pltpu.sync_copy(rows_v, acc.at[dst_v], add=True)
        return carry

    lax.fori_loop(0, CHUNKS_PER_W, chunk_body, 0)
    plsc.subcore_barrier()

    pltpu.sync_copy(acc.at[pl.ds(s * ZROWS, ZROWS)],
                    out_hbm.at[c, pl.ds(s * ZROWS, ZROWS)])


_sc_scatter = functools.partial(
    pl.kernel,
    out_type=jax.ShapeDtypeStruct((NC, ACC_ROWS, D), jnp.float32),
    mesh=plsc.VectorSubcoreMesh(core_axis_name="c", subcore_axis_name="s"),
    scratch_types=[
        pltpu.VMEM((CHUNK,), jnp.int32),
        pltpu.VMEM((CHUNK,), jnp.int32),
        pltpu.VMEM((CHUNK, D), jnp.float32),
        pltpu.VMEM_SHARED((ACC_ROWS, D), jnp.float32),
        pltpu.SemaphoreType.DMA,
    ],
)(_sc_scatter_body)


def _mix_body(h_ref, p0_ref, p1_ref, o_ref):
    o_ref[...] = ALPHA * h_ref[...] + (1.0 - ALPHA) * (p0_ref[...] + p1_ref[...])


def _mix(h, p0, p1):
    blk = 1000
    grid = N_NODES // blk
    spec = pl.BlockSpec((blk, D), lambda i: (i, 0))
    return pl.pallas_call(
        _mix_body,
        out_shape=jax.ShapeDtypeStruct((N_NODES, D), jnp.float32),
        grid=(grid,),
        in_specs=[spec, spec, spec],
        out_specs=spec,
    )(h, p0, p1)


def kernel(h, edge_index):
    src = edge_index[0].astype(jnp.int32)
    dst = edge_index[1].astype(jnp.int32)
    pad = E_PAD - N_EDGES
    src = jnp.concatenate([src, jnp.zeros((pad,), jnp.int32)])
    dst = jnp.concatenate([dst, jnp.full((pad,), N_NODES, jnp.int32)])
    src_rs = src.reshape(TOTAL_CHUNKS, CHUNK)
    dst_rs = dst.reshape(TOTAL_CHUNKS, CHUNK)
    zeros = jnp.zeros((ZROWS, D), jnp.float32)
    partials = _sc_scatter(h, src_rs, dst_rs, zeros)
    return _mix(h, partials[0, :N_NODES], partials[1, :N_NODES])


# speedup vs baseline: 3.5304x; 1.2113x over previous
import functools

import jax
import jax.numpy as jnp
from jax import lax
from jax.experimental import pallas as pl
from jax.experimental.pallas import tpu as pltpu
from jax.experimental.pallas import tpu_sc as plsc

ALPHA = 0.5
N_NODES = 10000
N_EDGES = 320000
D = 128

NC = 2
NS = 16
NW = NC * NS
CHUNK = 128
E_PAD = 327680
TOTAL_CHUNKS = E_PAD // CHUNK
CHUNKS_PER_W = TOTAL_CHUNKS // NW
ACC_ROWS = 10240
ZROWS = ACC_ROWS // NS


NBUF = 2
HALF_CHUNKS = CHUNKS_PER_W // 2


def _sc_scatter_body(h_hbm, src_hbm, dst_hbm, zeros_hbm, out_hbm,
                     src_all, dst_all, rows_v, acc, gsem0, gsem1):
    c = lax.axis_index("c")
    s = lax.axis_index("s")
    wid = s * NC + c
    base = wid * CHUNKS_PER_W

    pltpu.sync_copy(zeros_hbm, acc.at[pl.ds(s * ZROWS, ZROWS)])
    plsc.subcore_barrier()

    gsems = (gsem0, gsem1)
    for half in range(2):
        hbase = base + half * HALF_CHUNKS
        pltpu.sync_copy(src_hbm.at[pl.ds(hbase, HALF_CHUNKS)], src_all)
        pltpu.sync_copy(dst_hbm.at[pl.ds(hbase, HALF_CHUNKS)], dst_all)
        for b in range(NBUF):
            pltpu.async_copy(h_hbm.at[src_all.at[b]], rows_v.at[b], gsems[b])

        def group_body(g, carry):
            for b in range(NBUF):
                j = g * NBUF + b
                pltpu.make_async_copy(h_hbm.at[pl.ds(0, CHUNK)], rows_v.at[b],
                                      gsems[b]).wait()
                pltpu.sync_copy(rows_v.at[b], acc.at[dst_all.at[j]], add=True)

                @pl.when(j + NBUF < HALF_CHUNKS)
                def _():
                    pltpu.async_copy(h_hbm.at[src_all.at[j + NBUF]],
                                     rows_v.at[b], gsems[b])
            return carry

        lax.fori_loop(0, HALF_CHUNKS // NBUF, group_body, 0)
    plsc.subcore_barrier()

    pltpu.sync_copy(acc.at[pl.ds(s * ZROWS, ZROWS)],
                    out_hbm.at[c, pl.ds(s * ZROWS, ZROWS)])


_sc_scatter = functools.partial(
    pl.kernel,
    out_type=jax.ShapeDtypeStruct((NC, ACC_ROWS, D), jnp.float32),
    mesh=plsc.VectorSubcoreMesh(core_axis_name="c", subcore_axis_name="s"),
    scratch_types=[
        pltpu.VMEM((HALF_CHUNKS, CHUNK), jnp.int32),
        pltpu.VMEM((HALF_CHUNKS, CHUNK), jnp.int32),
        pltpu.VMEM((NBUF, CHUNK, D), jnp.float32),
        pltpu.VMEM_SHARED((ACC_ROWS, D), jnp.float32),
        pltpu.SemaphoreType.DMA,
        pltpu.SemaphoreType.DMA,
    ],
)(_sc_scatter_body)


def _mix_body(h_ref, p0_ref, p1_ref, o_ref):
    o_ref[...] = ALPHA * h_ref[...] + (1.0 - ALPHA) * (p0_ref[...] + p1_ref[...])


def _mix(h, p0, p1):
    blk = 1000
    grid = N_NODES // blk
    spec = pl.BlockSpec((blk, D), lambda i: (i, 0))
    return pl.pallas_call(
        _mix_body,
        out_shape=jax.ShapeDtypeStruct((N_NODES, D), jnp.float32),
        grid=(grid,),
        in_specs=[spec, spec, spec],
        out_specs=spec,
    )(h, p0, p1)


def kernel(h, edge_index):
    src = edge_index[0].astype(jnp.int32)
    dst = edge_index[1].astype(jnp.int32)
    pad = E_PAD - N_EDGES
    src = jnp.concatenate([src, jnp.zeros((pad,), jnp.int32)])
    dst = jnp.concatenate([dst, jnp.full((pad,), N_NODES, jnp.int32)])
    src_rs = src.reshape(TOTAL_CHUNKS, CHUNK)
    dst_rs = dst.reshape(TOTAL_CHUNKS, CHUNK)
    zeros = jnp.zeros((ZROWS, D), jnp.float32)
    partials = _sc_scatter(h, src_rs, dst_rs, zeros)
    return _mix(h, partials[0, :N_NODES], partials[1, :N_NODES])


# speedup vs baseline: 12.0445x; 3.4117x over previous
import functools

import jax
import jax.numpy as jnp
from jax import lax
from jax.experimental import pallas as pl
from jax.experimental.pallas import tpu as pltpu
from jax.experimental.pallas import tpu_sc as plsc

ALPHA = 0.5
N_NODES = 10000
N_EDGES = 320000
D = 128

NC = 2
NS = 16
NW = NC * NS
CHUNK = 128
E_PAD = 327680
TOTAL_CHUNKS = E_PAD // CHUNK
CHUNKS_PER_W = TOTAL_CHUNKS // NW
ACC_ROWS = 10240
ZROWS = ACC_ROWS // NS


NBUF = 2
HALF_CHUNKS = CHUNKS_PER_W // 2


def _sc_scatter_body(h_hbm, src_hbm, dst_hbm, zeros_hbm, out_hbm,
                     src_all, dst_all, rows_v, acc, gsem0, gsem1):
    c = lax.axis_index("c")
    s = lax.axis_index("s")
    wid = s * NC + c
    base = wid * CHUNKS_PER_W

    pltpu.sync_copy(zeros_hbm, acc.at[pl.ds(s * ZROWS, ZROWS)])
    plsc.subcore_barrier()

    gsems = (gsem0, gsem1)
    for half in range(2):
        hbase = base + half * HALF_CHUNKS
        pltpu.sync_copy(src_hbm.at[pl.ds(hbase, HALF_CHUNKS)], src_all)
        pltpu.sync_copy(dst_hbm.at[pl.ds(hbase, HALF_CHUNKS)], dst_all)
        for b in range(NBUF):
            pltpu.async_copy(h_hbm.at[src_all.at[b]], rows_v.at[b], gsems[b])

        def group_body(g, carry):
            for b in range(NBUF):
                j = g * NBUF + b
                pltpu.make_async_copy(h_hbm.at[pl.ds(0, CHUNK)], rows_v.at[b],
                                      gsems[b]).wait()
                pltpu.sync_copy(rows_v.at[b], acc.at[dst_all.at[j]], add=True)

                @pl.when(j + NBUF < HALF_CHUNKS)
                def _():
                    pltpu.async_copy(h_hbm.at[src_all.at[j + NBUF]],
                                     rows_v.at[b], gsems[b])
            return carry

        lax.fori_loop(0, HALF_CHUNKS // NBUF, group_body, 0)
    plsc.subcore_barrier()

    pltpu.sync_copy(acc.at[pl.ds(s * ZROWS, ZROWS)],
                    out_hbm.at[c, pl.ds(s * ZROWS, ZROWS)])


_sc_scatter = functools.partial(
    pl.kernel,
    out_type=jax.ShapeDtypeStruct((NC, ACC_ROWS, D), jnp.float32),
    mesh=plsc.VectorSubcoreMesh(core_axis_name="c", subcore_axis_name="s"),
    scratch_types=[
        pltpu.VMEM((HALF_CHUNKS, CHUNK), jnp.int32),
        pltpu.VMEM((HALF_CHUNKS, CHUNK), jnp.int32),
        pltpu.VMEM((NBUF, CHUNK, D), jnp.float32),
        pltpu.VMEM_SHARED((ACC_ROWS, D), jnp.float32),
        pltpu.SemaphoreType.DMA,
        pltpu.SemaphoreType.DMA,
    ],
)(_sc_scatter_body)


def _mix_body(h_ref, p0_ref, p1_ref, o_ref):
    o_ref[...] = ALPHA * h_ref[...] + (1.0 - ALPHA) * (p0_ref[...] + p1_ref[...])


def _mix(h, p0, p1):
    blk = 1000
    grid = N_NODES // blk
    spec = pl.BlockSpec((blk, D), lambda i: (i, 0))
    return pl.pallas_call(
        _mix_body,
        out_shape=jax.ShapeDtypeStruct((N_NODES, D), jnp.float32),
        grid=(grid,),
        in_specs=[spec, spec, spec],
        out_specs=spec,
    )(h, p0, p1)


def kernel(h, edge_index):
    src = edge_index[0].astype(jnp.int32)
    dst = edge_index[1].astype(jnp.int32)
    pad = E_PAD - N_EDGES
    iota = jnp.arange(pad, dtype=jnp.int32)
    src = jnp.concatenate([src, iota % N_NODES])
    dst = jnp.concatenate([dst, N_NODES + iota % (ACC_ROWS - N_NODES)])
    src_rs = src.reshape(TOTAL_CHUNKS, CHUNK)
    dst_rs = dst.reshape(TOTAL_CHUNKS, CHUNK)
    zeros = jnp.zeros((ZROWS, D), jnp.float32)
    partials = _sc_scatter(h, src_rs, dst_rs, zeros)
    return _mix(h, partials[0, :N_NODES], partials[1, :N_NODES])


# speedup vs baseline: 13.1334x; 1.0904x over previous
import functools

import jax
import jax.numpy as jnp
from jax import lax
from jax.experimental import pallas as pl
from jax.experimental.pallas import tpu as pltpu
from jax.experimental.pallas import tpu_sc as plsc

ALPHA = 0.5
N_NODES = 10000
N_EDGES = 320000
D = 128

NC = 2
NS = 16
NW = NC * NS
CHUNK = 125
TOTAL_CHUNKS = N_EDGES // CHUNK
CHUNKS_PER_W = TOTAL_CHUNKS // NW
ACC_ROWS = 10240
ZROWS = ACC_ROWS // NS
NBUF = 2
HALF_CHUNKS = CHUNKS_PER_W // 2


def _sc_scatter_body(h_hbm, edges_hbm, zeros_hbm, drain_hbm, out_hbm,
                     src_all, dst_all, rows_v, acc, gsem0, gsem1):
    c = lax.axis_index("c")
    s = lax.axis_index("s")
    wid = s * NC + c
    base = wid * CHUNKS_PER_W

    pltpu.sync_copy(zeros_hbm, acc.at[pl.ds(s * ZROWS, ZROWS)])
    plsc.subcore_barrier()

    gsems = (gsem0, gsem1)
    for half in range(2):
        hbase = base + half * HALF_CHUNKS
        pltpu.sync_copy(edges_hbm.at[0, pl.ds(hbase, HALF_CHUNKS)], src_all)
        pltpu.sync_copy(edges_hbm.at[1, pl.ds(hbase, HALF_CHUNKS)], dst_all)
        for b in range(NBUF):
            pltpu.async_copy(h_hbm.at[src_all.at[b]], rows_v.at[b], gsems[b])

        def group_body(g, carry):
            for b in range(NBUF):
                j = g * NBUF + b
                pltpu.make_async_copy(drain_hbm, rows_v.at[b],
                                      gsems[b]).wait()
                pltpu.sync_copy(rows_v.at[b], acc.at[dst_all.at[j]], add=True)

                @pl.when(j + NBUF < HALF_CHUNKS)
                def _():
                    pltpu.async_copy(h_hbm.at[src_all.at[j + NBUF]],
                                     rows_v.at[b], gsems[b])
            return carry

        lax.fori_loop(0, HALF_CHUNKS // NBUF, group_body, 0)
    plsc.subcore_barrier()

    pltpu.sync_copy(acc.at[pl.ds(s * ZROWS, ZROWS)],
                    out_hbm.at[c, pl.ds(s * ZROWS, ZROWS)])


_sc_scatter = functools.partial(
    pl.kernel,
    out_type=jax.ShapeDtypeStruct((NC, ACC_ROWS, D), jnp.float32),
    mesh=plsc.VectorSubcoreMesh(core_axis_name="c", subcore_axis_name="s"),
    scratch_types=[
        pltpu.VMEM((HALF_CHUNKS, CHUNK), jnp.int32),
        pltpu.VMEM((HALF_CHUNKS, CHUNK), jnp.int32),
        pltpu.VMEM((NBUF, CHUNK, D), jnp.float32),
        pltpu.VMEM_SHARED((ACC_ROWS, D), jnp.float32),
        pltpu.SemaphoreType.DMA,
        pltpu.SemaphoreType.DMA,
    ],
)(_sc_scatter_body)


def _mix_body(h_ref, p_ref, o_ref):
    o_ref[...] = ALPHA * h_ref[...] + (1.0 - ALPHA) * (p_ref[0] + p_ref[1])


def _mix(h, partials):
    blk = 1000
    grid = N_NODES // blk
    spec = pl.BlockSpec((blk, D), lambda i: (i, 0))
    return pl.pallas_call(
        _mix_body,
        out_shape=jax.ShapeDtypeStruct((N_NODES, D), jnp.float32),
        grid=(grid,),
        in_specs=[spec, pl.BlockSpec((NC, blk, D), lambda i: (0, i, 0))],
        out_specs=spec,
    )(h, partials)


def kernel(h, edge_index):
    edges = edge_index.astype(jnp.int32).reshape(2, TOTAL_CHUNKS, CHUNK)
    zeros = jnp.zeros((ZROWS, D), jnp.float32)
    drain = jnp.zeros((CHUNK, D), jnp.float32)
    partials = _sc_scatter(h, edges, zeros, drain)
    return _mix(h, partials)


# speedup vs baseline: 13.1501x; 1.0013x over previous
import functools

import jax
import jax.numpy as jnp
from jax import lax
from jax.experimental import pallas as pl
from jax.experimental.pallas import tpu as pltpu
from jax.experimental.pallas import tpu_sc as plsc

ALPHA = 0.5
N_NODES = 10000
N_EDGES = 320000
D = 128

NC = 2
NS = 16
NW = NC * NS
CHUNK = 125
TOTAL_CHUNKS = N_EDGES // CHUNK
CHUNKS_PER_W = TOTAL_CHUNKS // NW
ACC_ROWS = 10240
ZROWS = ACC_ROWS // NS
NBUF = 2
HALF_CHUNKS = CHUNKS_PER_W // 2


def _sc_scatter_body(h_hbm, edges_hbm, drain_hbm, out_hbm,
                     src_all, dst_all, rows_v, acc, gsem0, gsem1):
    c = lax.axis_index("c")
    s = lax.axis_index("s")
    wid = s * NC + c
    base = wid * CHUNKS_PER_W

    pltpu.sync_copy(edges_hbm.at[0, pl.ds(base, HALF_CHUNKS)], src_all)
    pltpu.sync_copy(edges_hbm.at[1, pl.ds(base, HALF_CHUNKS)], dst_all)
    zvec = jnp.zeros((16,), jnp.float32)

    def zstore(i, carry):
        rows_v[0, i // 8, pl.ds((i % 8) * 16, 16)] = zvec
        return carry

    lax.fori_loop(0, CHUNK * 8, zstore, 0)
    for k in range(ZROWS // CHUNK):
        pltpu.sync_copy(rows_v.at[0],
                        acc.at[pl.ds(s * ZROWS + k * CHUNK, CHUNK)])
    rem = ZROWS - (ZROWS // CHUNK) * CHUNK
    pltpu.sync_copy(rows_v.at[0, pl.ds(0, rem)],
                    acc.at[pl.ds(s * ZROWS + ZROWS - rem, rem)])
    plsc.subcore_barrier()

    gsems = (gsem0, gsem1)
    for half in range(2):
        hbase = base + half * HALF_CHUNKS
        if half > 0:
            pltpu.sync_copy(edges_hbm.at[0, pl.ds(hbase, HALF_CHUNKS)], src_all)
            pltpu.sync_copy(edges_hbm.at[1, pl.ds(hbase, HALF_CHUNKS)], dst_all)
        for b in range(NBUF):
            pltpu.async_copy(h_hbm.at[src_all.at[b]], rows_v.at[b], gsems[b])

        def group_body(g, carry):
            for b in range(NBUF):
                j = g * NBUF + b
                pltpu.make_async_copy(drain_hbm, rows_v.at[b],
                                      gsems[b]).wait()
                pltpu.sync_copy(rows_v.at[b], acc.at[dst_all.at[j]], add=True)

                @pl.when(j + NBUF < HALF_CHUNKS)
                def _():
                    pltpu.async_copy(h_hbm.at[src_all.at[j + NBUF]],
                                     rows_v.at[b], gsems[b])
            return carry

        lax.fori_loop(0, HALF_CHUNKS // NBUF, group_body, 0)
    plsc.subcore_barrier()

    pltpu.sync_copy(acc.at[pl.ds(s * ZROWS, ZROWS)],
                    out_hbm.at[c, pl.ds(s * ZROWS, ZROWS)])


_sc_scatter = functools.partial(
    pl.kernel,
    out_type=jax.ShapeDtypeStruct((NC, ACC_ROWS, D), jnp.float32),
    mesh=plsc.VectorSubcoreMesh(core_axis_name="c", subcore_axis_name="s"),
    scratch_types=[
        pltpu.VMEM((HALF_CHUNKS, CHUNK), jnp.int32),
        pltpu.VMEM((HALF_CHUNKS, CHUNK), jnp.int32),
        pltpu.VMEM((NBUF, CHUNK, D), jnp.float32),
        pltpu.VMEM_SHARED((ACC_ROWS, D), jnp.float32),
        pltpu.SemaphoreType.DMA,
        pltpu.SemaphoreType.DMA,
    ],
)(_sc_scatter_body)


def _mix_body(h_ref, p_ref, o_ref):
    o_ref[...] = ALPHA * h_ref[...] + (1.0 - ALPHA) * (p_ref[0] + p_ref[1])


def _mix(h, partials):
    blk = 1000
    grid = N_NODES // blk
    spec = pl.BlockSpec((blk, D), lambda i: (i, 0))
    return pl.pallas_call(
        _mix_body,
        out_shape=jax.ShapeDtypeStruct((N_NODES, D), jnp.float32),
        grid=(grid,),
        in_specs=[spec, pl.BlockSpec((NC, blk, D), lambda i: (0, i, 0))],
        out_specs=spec,
    )(h, partials)


def kernel(h, edge_index):
    edges = edge_index.astype(jnp.int32).reshape(2, TOTAL_CHUNKS, CHUNK)
    drain = jnp.zeros((CHUNK, D), jnp.float32)
    partials = _sc_scatter(h, edges, drain)
    return _mix(h, partials)
